# baseline (speedup 1.0000x reference)
import jax
import jax.numpy as jnp
from jax.experimental import pallas as pl
from jax.experimental.pallas import tpu as pltpu

_BATCH = 16
_NODE = 32
_DEGREE = 64
_IN_F = 128
_OUT_F = 3
_N = _NODE * _DEGREE
_K = 8
_ROWS = 256
_COUNTS = (1, 2, 4, 8, 16, 32)


def _leaky(x):
    return jnp.where(x >= 0, x, 0.2 * x)


def _bdot(a, b):
    return jnp.dot(a.astype(jnp.bfloat16), b.astype(jnp.bfloat16),
                   preferred_element_type=jnp.float32)


def _stage1_body(t0, t1, t2, t3, t4, t5,
                 wr0, wr1, wr2, wr3, wr4, wr5,
                 wb, wl1, wl2, x_out):
    n = pl.program_id(0)
    trees = (t0, t1, t2, t3, t4, t5)
    wroots = (wr0, wr1, wr2, wr3, wr4, wr5)

    root_n = jnp.zeros((_BATCH, _OUT_F), dtype=jnp.float32)
    sel_row = None
    for i in range(6):
        cnt = _COUNTS[i]
        rep = _NODE // cnt
        idx = n // rep
        tv = trees[i][...]
        cids = jax.lax.broadcasted_iota(jnp.int32, tv.shape, 1)
        row = jnp.sum(jnp.where(cids == idx, tv, 0.0), axis=1)
        if i == 5:
            sel_row = row
        root_n = root_n + _bdot(row, wroots[i][...])

    bmat = _bdot(sel_row, wb[0])
    bmat = _leaky(bmat)
    bmat = bmat.reshape(_BATCH * _DEGREE, _IN_F)
    h = _bdot(bmat, wl1[...])
    q = _bdot(h, wl2[...])
    q = q.reshape(_BATCH, _DEGREE, _OUT_F)
    x_out[...] = q + root_n[:, None, :]


def _stage2_body(x_ref, c1w, c1b, c2w, c2b, bias_ref, out_ref, kscr, vscr):
    r = pl.program_id(1)

    xb = x_ref[0]
    xr = x_ref[0, pl.ds(r * _ROWS, _ROWS), :]

    w12 = _bdot(c1w[...], c2w[...])
    w1 = w12[0:3, :]
    wd = w12[3:6, :] - w1
    b3 = _bdot(c1b[...], c2w[...]) + c2b[...]

    pt = jax.lax.dot_general(w1, xb, (((0,), (1,)), ((), ())),
                             preferred_element_type=jnp.float32,
                             precision=jax.lax.Precision.HIGHEST)

    g = jax.lax.dot_general(xr.astype(jnp.bfloat16), xb.astype(jnp.bfloat16),
                            (((1,), (1,)), ((), ())),
                            preferred_element_type=jnp.float32)
    inner = -2.0 * g
    sq_col = jnp.sum(xr * xr, axis=1, keepdims=True)
    sq_row = jax.lax.dot_general(jnp.ones((1, 3), jnp.float32), xb * xb,
                                 (((1,), (1,)), ((), ())),
                                 preferred_element_type=jnp.float32,
                                 precision=jax.lax.Precision.HIGHEST)
    kscr[...] = (-sq_row) - inner - sq_col

    neg_inf = jnp.float32(-jnp.inf)

    def strip(s, carry):
        k8 = kscr[pl.ds(s * 8, 8), :]
        m = jnp.max(k8, axis=1, keepdims=True)
        for _ in range(_K - 1):
            m = jnp.max(jnp.where(k8 < m, k8, neg_inf), axis=1, keepdims=True)
        mask8 = k8 >= m
        vs = [jnp.max(jnp.where(mask8, pt[o:o + 1, :], neg_inf),
                      axis=1, keepdims=True) for o in range(_OUT_F)]
        vscr[pl.ds(s * 8, 8), :] = jnp.concatenate(vs, axis=1)
        return carry

    jax.lax.fori_loop(0, _ROWS // 8, strip, 0, unroll=False)

    cr = _bdot(xr, wd) + b3
    bias_tile = jnp.concatenate([bias_ref[...]] * (_ROWS // _DEGREE), axis=0)
    o = vscr[...] + cr + bias_tile
    out_ref[0] = _leaky(o)


@jax.jit
def kernel(tree0, tree1, tree2, tree3, tree4, tree5,
           W_root0, W_root1, W_root2, W_root3, W_root4, W_root5,
           W_branch, W_loop1, W_loop2, bias,
           conv1_w, conv1_b, conv2_w, conv2_b):
    full = lambda s: pl.BlockSpec(s, lambda n: (0,) * len(s))
    x = pl.pallas_call(
        _stage1_body,
        grid=(_NODE,),
        in_specs=[
            full(tree0.shape), full(tree1.shape), full(tree2.shape),
            full(tree3.shape), full(tree4.shape), full(tree5.shape),
            full(W_root0.shape), full(W_root1.shape), full(W_root2.shape),
            full(W_root3.shape), full(W_root4.shape), full(W_root5.shape),
            pl.BlockSpec((1, _IN_F, _DEGREE * _IN_F), lambda n: (n, 0, 0)),
            full(W_loop1.shape), full(W_loop2.shape),
        ],
        out_specs=pl.BlockSpec((_BATCH, _DEGREE, _OUT_F), lambda n: (0, n, 0)),
        out_shape=jax.ShapeDtypeStruct((_BATCH, _N, _OUT_F), jnp.float32),
        compiler_params=pltpu.CompilerParams(
            dimension_semantics=("arbitrary",),
        ),
    )(tree0, tree1, tree2, tree3, tree4, tree5,
      W_root0, W_root1, W_root2, W_root3, W_root4, W_root5,
      W_branch, W_loop1, W_loop2)

    full2 = lambda s: pl.BlockSpec(s, lambda b, r: (0,) * len(s))
    out = pl.pallas_call(
        _stage2_body,
        grid=(_BATCH, _N // _ROWS),
        in_specs=[
            pl.BlockSpec((1, _N, _OUT_F), lambda b, r: (b, 0, 0)),
            full2(conv1_w.shape),
            full2((1, 64)),
            full2(conv2_w.shape),
            full2((1, 3)),
            full2((_DEGREE, _OUT_F)),
        ],
        out_specs=pl.BlockSpec((1, _ROWS, _OUT_F), lambda b, r: (b, r, 0)),
        out_shape=jax.ShapeDtypeStruct((_BATCH, _N, _OUT_F), jnp.float32),
        scratch_shapes=[
            pltpu.VMEM((_ROWS, _N), jnp.float32),
            pltpu.VMEM((_ROWS, _OUT_F), jnp.float32),
        ],
        compiler_params=pltpu.CompilerParams(
            dimension_semantics=("arbitrary", "arbitrary"),
        ),
    )(x, conv1_w, conv1_b.reshape(1, 64), conv2_w, conv2_b.reshape(1, 3),
      bias.reshape(_DEGREE, _OUT_F))
    return out


# speedup vs baseline: 3.9665x; 3.9665x over previous
import jax
import jax.numpy as jnp
from jax.experimental import pallas as pl
from jax.experimental.pallas import tpu as pltpu

_BATCH = 16
_NODE = 32
_DEGREE = 64
_IN_F = 128
_OUT_F = 3
_N = _NODE * _DEGREE
_K = 8
_ROWS = 256
_COUNTS = (1, 2, 4, 8, 16, 32)


def _leaky(x):
    return jnp.where(x >= 0, x, 0.2 * x)


def _bdot(a, b):
    return jnp.dot(a.astype(jnp.bfloat16), b.astype(jnp.bfloat16),
                   preferred_element_type=jnp.float32)


def _stage1_body(t0, t1, t2, t3, t4, t5,
                 wr0, wr1, wr2, wr3, wr4, wr5,
                 wb, wl1, wl2, x_out):
    n = pl.program_id(0)
    trees = (t0, t1, t2, t3, t4, t5)
    wroots = (wr0, wr1, wr2, wr3, wr4, wr5)

    root_n = jnp.zeros((_BATCH, _OUT_F), dtype=jnp.float32)
    sel_row = None
    for i in range(6):
        cnt = _COUNTS[i]
        rep = _NODE // cnt
        idx = n // rep
        tv = trees[i][...]
        cids = jax.lax.broadcasted_iota(jnp.int32, tv.shape, 1)
        row = jnp.sum(jnp.where(cids == idx, tv, 0.0), axis=1)
        if i == 5:
            sel_row = row
        root_n = root_n + _bdot(row, wroots[i][...])

    bmat = _bdot(sel_row, wb[0])
    bmat = _leaky(bmat)
    bmat = bmat.reshape(_BATCH * _DEGREE, _IN_F)
    h = _bdot(bmat, wl1[...])
    q = _bdot(h, wl2[...])
    q = q.reshape(_BATCH, _DEGREE, _OUT_F)
    x_out[...] = q + root_n[:, None, :]


def _stage2_body(x_ref, c1w, c1b, c2w, c2b, bias_ref, out_ref, kscr, vscr):
    r = pl.program_id(1)

    xb = x_ref[0]
    xr = x_ref[0, pl.ds(r * _ROWS, _ROWS), :]

    w12 = _bdot(c1w[...], c2w[...])
    w1 = w12[0:3, :]
    wd = w12[3:6, :] - w1
    b3 = _bdot(c1b[...], c2w[...]) + c2b[...]

    pt = jax.lax.dot_general(w1, xb, (((0,), (1,)), ((), ())),
                             preferred_element_type=jnp.float32,
                             precision=jax.lax.Precision.HIGHEST)

    g = jax.lax.dot_general(xr.astype(jnp.bfloat16), xb.astype(jnp.bfloat16),
                            (((1,), (1,)), ((), ())),
                            preferred_element_type=jnp.float32)
    inner = -2.0 * g
    sq_col = jnp.sum(xr * xr, axis=1, keepdims=True)
    sq_row = jax.lax.dot_general(jnp.ones((1, 3), jnp.float32), xb * xb,
                                 (((1,), (1,)), ((), ())),
                                 preferred_element_type=jnp.float32,
                                 precision=jax.lax.Precision.HIGHEST)
    kscr[...] = (-sq_row) - inner - sq_col

    neg_inf = jnp.float32(-jnp.inf)

    def strip(s, carry):
        k8 = kscr[pl.ds(s * 8, 8), :]
        m = jnp.max(k8, axis=1, keepdims=True)
        for _ in range(_K - 1):
            m = jnp.max(jnp.where(k8 < m, k8, neg_inf), axis=1, keepdims=True)
        mask8 = k8 >= m
        vs = [jnp.max(jnp.where(mask8, pt[o:o + 1, :], neg_inf),
                      axis=1, keepdims=True) for o in range(_OUT_F)]
        vscr[pl.ds(s * 8, 8), :] = jnp.concatenate(vs, axis=1)
        return carry

    jax.lax.fori_loop(0, _ROWS // 8, strip, 0, unroll=8)

    cr = _bdot(xr, wd) + b3
    bias_tile = jnp.concatenate([bias_ref[...]] * (_ROWS // _DEGREE), axis=0)
    o = vscr[...] + cr + bias_tile
    out_ref[0] = _leaky(o)


@jax.jit
def kernel(tree0, tree1, tree2, tree3, tree4, tree5,
           W_root0, W_root1, W_root2, W_root3, W_root4, W_root5,
           W_branch, W_loop1, W_loop2, bias,
           conv1_w, conv1_b, conv2_w, conv2_b):
    full = lambda s: pl.BlockSpec(s, lambda n: (0,) * len(s))
    x = pl.pallas_call(
        _stage1_body,
        grid=(_NODE,),
        in_specs=[
            full(tree0.shape), full(tree1.shape), full(tree2.shape),
            full(tree3.shape), full(tree4.shape), full(tree5.shape),
            full(W_root0.shape), full(W_root1.shape), full(W_root2.shape),
            full(W_root3.shape), full(W_root4.shape), full(W_root5.shape),
            pl.BlockSpec((1, _IN_F, _DEGREE * _IN_F), lambda n: (n, 0, 0)),
            full(W_loop1.shape), full(W_loop2.shape),
        ],
        out_specs=pl.BlockSpec((_BATCH, _DEGREE, _OUT_F), lambda n: (0, n, 0)),
        out_shape=jax.ShapeDtypeStruct((_BATCH, _N, _OUT_F), jnp.float32),
        compiler_params=pltpu.CompilerParams(
            dimension_semantics=("arbitrary",),
        ),
    )(tree0, tree1, tree2, tree3, tree4, tree5,
      W_root0, W_root1, W_root2, W_root3, W_root4, W_root5,
      W_branch, W_loop1, W_loop2)

    full2 = lambda s: pl.BlockSpec(s, lambda b, r: (0,) * len(s))
    out = pl.pallas_call(
        _stage2_body,
        grid=(_BATCH, _N // _ROWS),
        in_specs=[
            pl.BlockSpec((1, _N, _OUT_F), lambda b, r: (b, 0, 0)),
            full2(conv1_w.shape),
            full2((1, 64)),
            full2(conv2_w.shape),
            full2((1, 3)),
            full2((_DEGREE, _OUT_F)),
        ],
        out_specs=pl.BlockSpec((1, _ROWS, _OUT_F), lambda b, r: (b, r, 0)),
        out_shape=jax.ShapeDtypeStruct((_BATCH, _N, _OUT_F), jnp.float32),
        scratch_shapes=[
            pltpu.VMEM((_ROWS, _N), jnp.float32),
            pltpu.VMEM((_ROWS, _OUT_F), jnp.float32),
        ],
        compiler_params=pltpu.CompilerParams(
            dimension_semantics=("arbitrary", "arbitrary"),
        ),
    )(x, conv1_w, conv1_b.reshape(1, 64), conv2_w, conv2_b.reshape(1, 3),
      bias.reshape(_DEGREE, _OUT_F))
    return out


# speedup vs baseline: 7.9127x; 1.9949x over previous
import jax
import jax.numpy as jnp
from jax.experimental import pallas as pl
from jax.experimental.pallas import tpu as pltpu

_BATCH = 16
_NODE = 32
_DEGREE = 64
_IN_F = 128
_OUT_F = 3
_N = _NODE * _DEGREE
_K = 8
_ROWS = 256
_COUNTS = (1, 2, 4, 8, 16, 32)


def _leaky(x):
    return jnp.where(x >= 0, x, 0.2 * x)


def _bdot(a, b):
    return jnp.dot(a.astype(jnp.bfloat16), b.astype(jnp.bfloat16),
                   preferred_element_type=jnp.float32)


def _stage1_body(t0, t1, t2, t3, t4, t5,
                 wr0, wr1, wr2, wr3, wr4, wr5,
                 wb, wl1, wl2, x_out):
    n = pl.program_id(0)
    trees = (t0, t1, t2, t3, t4, t5)
    wroots = (wr0, wr1, wr2, wr3, wr4, wr5)

    root_n = jnp.zeros((_BATCH, _OUT_F), dtype=jnp.float32)
    sel_row = None
    for i in range(6):
        cnt = _COUNTS[i]
        rep = _NODE // cnt
        idx = n // rep
        tv = trees[i][...]
        cids = jax.lax.broadcasted_iota(jnp.int32, tv.shape, 1)
        row = jnp.sum(jnp.where(cids == idx, tv, 0.0), axis=1)
        if i == 5:
            sel_row = row
        root_n = root_n + _bdot(row, wroots[i][...])

    bmat = _bdot(sel_row, wb[0])
    bmat = _leaky(bmat)
    bmat = bmat.reshape(_BATCH * _DEGREE, _IN_F)
    h = _bdot(bmat, wl1[...])
    q = _bdot(h, wl2[...])
    q = q.reshape(_BATCH, _DEGREE, _OUT_F)
    x_out[...] = q + root_n[:, None, :]


def _stage2_body(x_ref, c1w, c1b, c2w, c2b, bias_ref, out_ref, aux):
    r = pl.program_id(1)

    xb = x_ref[0]
    xr = x_ref[0, pl.ds(r * _ROWS, _ROWS), :]

    w12 = _bdot(c1w[...], c2w[...])
    w1 = w12[0:3, :]
    wd = w12[3:6, :] - w1
    b3 = _bdot(c1b[...], c2w[...]) + c2b[...]

    @pl.when(r == 0)
    def _hoist():
        aux[0:3, :] = jax.lax.dot_general(w1, xb, (((0,), (1,)), ((), ())),
                                          preferred_element_type=jnp.float32,
                                          precision=jax.lax.Precision.HIGHEST)
        aux[3:4, :] = jax.lax.dot_general(jnp.ones((1, 3), jnp.float32), xb * xb,
                                          (((1,), (1,)), ((), ())),
                                          preferred_element_type=jnp.float32,
                                          precision=jax.lax.Precision.HIGHEST)

    pt = aux[0:3, :]
    sq_row = aux[3:4, :]

    g = jax.lax.dot_general(xr.astype(jnp.bfloat16), xb.astype(jnp.bfloat16),
                            (((1,), (1,)), ((), ())),
                            preferred_element_type=jnp.float32)
    inner = -2.0 * g
    sq_col = jnp.sum(xr * xr, axis=1, keepdims=True)
    key0 = (-sq_row) - inner - sq_col

    neg_inf = jnp.float32(-jnp.inf)

    sl = [key0[:, gg * 128:(gg + 1) * 128] for gg in range(16)]

    def cswap(a, b):
        return jnp.maximum(a, b), jnp.minimum(a, b)

    def sort8(v):
        net = [(0, 1), (2, 3), (4, 5), (6, 7),
               (0, 2), (1, 3), (4, 6), (5, 7),
               (1, 2), (5, 6), (0, 4), (3, 7),
               (1, 5), (2, 6),
               (1, 4), (3, 6),
               (2, 4), (3, 5),
               (3, 4)]
        v = list(v)
        for a, b in net:
            v[a], v[b] = cswap(v[a], v[b])
        return v

    lo = sort8(sl[0:8])
    hi = sort8(sl[8:16])
    top = [jnp.maximum(lo[i], hi[7 - i]) for i in range(8)]
    cand = jnp.concatenate(top, axis=1)

    m = None
    for t in range(_K):
        m = jnp.max(cand, axis=1, keepdims=True)
        if t < _K - 1:
            cand = jnp.where(cand == m, neg_inf, cand)
    mask8 = key0 >= m

    vs = [jnp.max(jnp.where(mask8, pt[o:o + 1, :], neg_inf), axis=1, keepdims=True)
          for o in range(_OUT_F)]
    v3 = jnp.concatenate(vs, axis=1)

    cr = _bdot(xr, wd) + b3
    bias_tile = jnp.concatenate([bias_ref[...]] * (_ROWS // _DEGREE), axis=0)
    o = v3 + cr + bias_tile
    out_ref[0] = _leaky(o)


@jax.jit
def kernel(tree0, tree1, tree2, tree3, tree4, tree5,
           W_root0, W_root1, W_root2, W_root3, W_root4, W_root5,
           W_branch, W_loop1, W_loop2, bias,
           conv1_w, conv1_b, conv2_w, conv2_b):
    full = lambda s: pl.BlockSpec(s, lambda n: (0,) * len(s))
    x = pl.pallas_call(
        _stage1_body,
        grid=(_NODE,),
        in_specs=[
            full(tree0.shape), full(tree1.shape), full(tree2.shape),
            full(tree3.shape), full(tree4.shape), full(tree5.shape),
            full(W_root0.shape), full(W_root1.shape), full(W_root2.shape),
            full(W_root3.shape), full(W_root4.shape), full(W_root5.shape),
            pl.BlockSpec((1, _IN_F, _DEGREE * _IN_F), lambda n: (n, 0, 0)),
            full(W_loop1.shape), full(W_loop2.shape),
        ],
        out_specs=pl.BlockSpec((_BATCH, _DEGREE, _OUT_F), lambda n: (0, n, 0)),
        out_shape=jax.ShapeDtypeStruct((_BATCH, _N, _OUT_F), jnp.float32),
        compiler_params=pltpu.CompilerParams(
            dimension_semantics=("arbitrary",),
        ),
    )(tree0, tree1, tree2, tree3, tree4, tree5,
      W_root0, W_root1, W_root2, W_root3, W_root4, W_root5,
      W_branch, W_loop1, W_loop2)

    full2 = lambda s: pl.BlockSpec(s, lambda b, r: (0,) * len(s))
    out = pl.pallas_call(
        _stage2_body,
        grid=(_BATCH, _N // _ROWS),
        in_specs=[
            pl.BlockSpec((1, _N, _OUT_F), lambda b, r: (b, 0, 0)),
            full2(conv1_w.shape),
            full2((1, 64)),
            full2(conv2_w.shape),
            full2((1, 3)),
            full2((_DEGREE, _OUT_F)),
        ],
        out_specs=pl.BlockSpec((1, _ROWS, _OUT_F), lambda b, r: (b, r, 0)),
        out_shape=jax.ShapeDtypeStruct((_BATCH, _N, _OUT_F), jnp.float32),
        scratch_shapes=[
            pltpu.VMEM((8, _N), jnp.float32),
        ],
        compiler_params=pltpu.CompilerParams(
            dimension_semantics=("arbitrary", "arbitrary"),
        ),
    )(x, conv1_w, conv1_b.reshape(1, 64), conv2_w, conv2_b.reshape(1, 3),
      bias.reshape(_DEGREE, _OUT_F))
    return out


# speedup vs baseline: 7.9695x; 1.0072x over previous
import jax
import jax.numpy as jnp
from jax.experimental import pallas as pl
from jax.experimental.pallas import tpu as pltpu

_BATCH = 16
_NODE = 32
_DEGREE = 64
_IN_F = 128
_OUT_F = 3
_N = _NODE * _DEGREE
_K = 8
_ROWS = 512
_COUNTS = (1, 2, 4, 8, 16, 32)


def _leaky(x):
    return jnp.where(x >= 0, x, 0.2 * x)


def _bdot(a, b):
    return jnp.dot(a.astype(jnp.bfloat16), b.astype(jnp.bfloat16),
                   preferred_element_type=jnp.float32)


_HALF = _DEGREE * _IN_F // 2


def _stage1_body(t0, t1, t2, t3, t4, t5,
                 wr0, wr1, wr2, wr3, wr4, wr5,
                 wb, wl1, wl2, x_out, root_scr):
    n = pl.program_id(0)
    j = pl.program_id(1)
    trees = (t0, t1, t2, t3, t4, t5)
    wroots = (wr0, wr1, wr2, wr3, wr4, wr5)

    def sel(i):
        cnt = _COUNTS[i]
        rep = _NODE // cnt
        idx = n // rep
        tv = trees[i][...]
        cids = jax.lax.broadcasted_iota(jnp.int32, tv.shape, 1)
        return jnp.sum(jnp.where(cids == idx, tv, 0.0), axis=1)

    @pl.when(j == 0)
    def _root():
        root_n = jnp.zeros((_BATCH, _OUT_F), dtype=jnp.float32)
        for i in range(6):
            root_n = root_n + _bdot(sel(i), wroots[i][...])
        root_scr[...] = root_n

    bmat = _bdot(sel(5), wb[0])
    bmat = _leaky(bmat)
    bmat = bmat.reshape(_BATCH * _DEGREE // 2, _IN_F)
    h = _bdot(bmat, wl1[...])
    q = _bdot(h, wl2[...])
    q = q.reshape(_BATCH, _DEGREE // 2, _OUT_F)
    x_out[...] = q + root_scr[...][:, None, :]


def _stage2_body(x_ref, c1w, c1b, c2w, c2b, bias_ref, out_ref, aux):
    r = pl.program_id(1)

    xb = x_ref[0]
    xr = x_ref[0, pl.ds(r * _ROWS, _ROWS), :]

    w12 = _bdot(c1w[...], c2w[...])
    w1 = w12[0:3, :]
    wd = w12[3:6, :] - w1
    b3 = _bdot(c1b[...], c2w[...]) + c2b[...]

    @pl.when(r == 0)
    def _hoist():
        aux[0:3, :] = jax.lax.dot_general(w1, xb, (((0,), (1,)), ((), ())),
                                          preferred_element_type=jnp.float32,
                                          precision=jax.lax.Precision.HIGHEST)
        aux[3:4, :] = jax.lax.dot_general(jnp.ones((1, 3), jnp.float32), xb * xb,
                                          (((1,), (1,)), ((), ())),
                                          preferred_element_type=jnp.float32,
                                          precision=jax.lax.Precision.HIGHEST)

    pt = aux[0:3, :]
    sq_row = aux[3:4, :]

    g = jax.lax.dot_general(xr.astype(jnp.bfloat16), xb.astype(jnp.bfloat16),
                            (((1,), (1,)), ((), ())),
                            preferred_element_type=jnp.float32)
    inner = -2.0 * g
    sq_col = jnp.sum(xr * xr, axis=1, keepdims=True)
    key0 = (-sq_row) - inner - sq_col

    neg_inf = jnp.float32(-jnp.inf)

    sl = [key0[:, gg * 128:(gg + 1) * 128] for gg in range(16)]

    def cswap(a, b):
        return jnp.maximum(a, b), jnp.minimum(a, b)

    def sort8(v):
        net = [(0, 1), (2, 3), (4, 5), (6, 7),
               (0, 2), (1, 3), (4, 6), (5, 7),
               (1, 2), (5, 6), (0, 4), (3, 7),
               (1, 5), (2, 6),
               (1, 4), (3, 6),
               (2, 4), (3, 5),
               (3, 4)]
        v = list(v)
        for a, b in net:
            v[a], v[b] = cswap(v[a], v[b])
        return v

    lo = sort8(sl[0:8])
    hi = sort8(sl[8:16])
    top = [jnp.maximum(lo[i], hi[7 - i]) for i in range(8)]
    cand = jnp.concatenate(top, axis=1)

    m = None
    for t in range(_K):
        m = jnp.max(cand, axis=1, keepdims=True)
        if t < _K - 1:
            cand = jnp.where(cand == m, neg_inf, cand)
    mask8 = key0 >= m

    vs = [jnp.max(jnp.where(mask8, pt[o:o + 1, :], neg_inf), axis=1, keepdims=True)
          for o in range(_OUT_F)]
    v3 = jnp.concatenate(vs, axis=1)

    cr = _bdot(xr, wd) + b3
    bias_tile = jnp.concatenate([bias_ref[...]] * (_ROWS // _DEGREE), axis=0)
    o = v3 + cr + bias_tile
    out_ref[0] = _leaky(o)


@jax.jit
def kernel(tree0, tree1, tree2, tree3, tree4, tree5,
           W_root0, W_root1, W_root2, W_root3, W_root4, W_root5,
           W_branch, W_loop1, W_loop2, bias,
           conv1_w, conv1_b, conv2_w, conv2_b):
    full = lambda s: pl.BlockSpec(s, lambda n: (0,) * len(s))
    full1 = lambda s: pl.BlockSpec(s, lambda n, j: (0,) * len(s))
    x = pl.pallas_call(
        _stage1_body,
        grid=(_NODE, 2),
        in_specs=[
            full1(tree0.shape), full1(tree1.shape), full1(tree2.shape),
            full1(tree3.shape), full1(tree4.shape), full1(tree5.shape),
            full1(W_root0.shape), full1(W_root1.shape), full1(W_root2.shape),
            full1(W_root3.shape), full1(W_root4.shape), full1(W_root5.shape),
            pl.BlockSpec((1, _IN_F, _HALF), lambda n, j: (n, 0, j)),
            full1(W_loop1.shape), full1(W_loop2.shape),
        ],
        out_specs=pl.BlockSpec((_BATCH, _DEGREE // 2, _OUT_F),
                               lambda n, j: (0, 2 * n + j, 0)),
        out_shape=jax.ShapeDtypeStruct((_BATCH, _N, _OUT_F), jnp.float32),
        scratch_shapes=[pltpu.VMEM((_BATCH, _OUT_F), jnp.float32)],
        compiler_params=pltpu.CompilerParams(
            dimension_semantics=("arbitrary", "arbitrary"),
        ),
    )(tree0, tree1, tree2, tree3, tree4, tree5,
      W_root0, W_root1, W_root2, W_root3, W_root4, W_root5,
      W_branch, W_loop1, W_loop2)

    full2 = lambda s: pl.BlockSpec(s, lambda b, r: (0,) * len(s))
    out = pl.pallas_call(
        _stage2_body,
        grid=(_BATCH, _N // _ROWS),
        in_specs=[
            pl.BlockSpec((1, _N, _OUT_F), lambda b, r: (b, 0, 0)),
            full2(conv1_w.shape),
            full2((1, 64)),
            full2(conv2_w.shape),
            full2((1, 3)),
            full2((_DEGREE, _OUT_F)),
        ],
        out_specs=pl.BlockSpec((1, _ROWS, _OUT_F), lambda b, r: (b, r, 0)),
        out_shape=jax.ShapeDtypeStruct((_BATCH, _N, _OUT_F), jnp.float32),
        scratch_shapes=[
            pltpu.VMEM((8, _N), jnp.float32),
        ],
        compiler_params=pltpu.CompilerParams(
            dimension_semantics=("arbitrary", "arbitrary"),
        ),
    )(x, conv1_w, conv1_b.reshape(1, 64), conv2_w, conv2_b.reshape(1, 3),
      bias.reshape(_DEGREE, _OUT_F))
    return out


# speedup vs baseline: 8.7381x; 1.0964x over previous
import jax
import jax.numpy as jnp
from jax.experimental import pallas as pl
from jax.experimental.pallas import tpu as pltpu

_BATCH = 16
_NODE = 32
_DEGREE = 64
_IN_F = 128
_OUT_F = 3
_N = _NODE * _DEGREE
_K = 8
_ROWS = 1024
_COUNTS = (1, 2, 4, 8, 16, 32)


def _leaky(x):
    return jnp.where(x >= 0, x, 0.2 * x)


def _bdot(a, b):
    return jnp.dot(a.astype(jnp.bfloat16), b.astype(jnp.bfloat16),
                   preferred_element_type=jnp.float32)


def _stage1_body(t0, t1, t2, t3, t4, t5,
                 wr0, wr1, wr2, wr3, wr4, wr5,
                 wb, wl1, wl2, x_out):
    n = pl.program_id(0)
    trees = (t0, t1, t2, t3, t4, t5)
    wroots = (wr0, wr1, wr2, wr3, wr4, wr5)

    root_n = jnp.zeros((_BATCH, _OUT_F), dtype=jnp.float32)
    sel_row = None
    for i in range(6):
        cnt = _COUNTS[i]
        rep = _NODE // cnt
        idx = n // rep
        tv = trees[i][...]
        cids = jax.lax.broadcasted_iota(jnp.int32, tv.shape, 1)
        row = jnp.sum(jnp.where(cids == idx, tv, 0.0), axis=1)
        if i == 5:
            sel_row = row
        root_n = root_n + _bdot(row, wroots[i][...])

    bmat = _bdot(sel_row, wb[0])
    bmat = _leaky(bmat)
    bmat = bmat.reshape(_BATCH * _DEGREE, _IN_F)
    h = _bdot(bmat, wl1[...])
    q = _bdot(h, wl2[...])
    q = q.reshape(_BATCH, _DEGREE, _OUT_F)
    x_out[...] = q + root_n[:, None, :]


def _stage2_body(x_ref, c1w, c1b, c2w, c2b, bias_ref, out_ref, aux):
    r = pl.program_id(1)

    xb = x_ref[0]
    xr = x_ref[0, pl.ds(r * _ROWS, _ROWS), :]

    w12 = _bdot(c1w[...], c2w[...])
    w1 = w12[0:3, :]
    wd = w12[3:6, :] - w1
    b3 = _bdot(c1b[...], c2w[...]) + c2b[...]

    @pl.when(r == 0)
    def _hoist():
        aux[0:3, :] = jax.lax.dot_general(w1, xb, (((0,), (1,)), ((), ())),
                                          preferred_element_type=jnp.float32,
                                          precision=jax.lax.Precision.HIGHEST)
        aux[3:4, :] = jax.lax.dot_general(jnp.ones((1, 3), jnp.float32), xb * xb,
                                          (((1,), (1,)), ((), ())),
                                          preferred_element_type=jnp.float32,
                                          precision=jax.lax.Precision.HIGHEST)

    pt = aux[0:3, :]
    sq_row = aux[3:4, :]

    g = jax.lax.dot_general(xr.astype(jnp.bfloat16), xb.astype(jnp.bfloat16),
                            (((1,), (1,)), ((), ())),
                            preferred_element_type=jnp.float32)
    inner = -2.0 * g
    sq_col = jnp.sum(xr * xr, axis=1, keepdims=True)
    key0 = (-sq_row) - inner - sq_col

    neg_inf = jnp.float32(-jnp.inf)

    sl = [key0[:, gg * 128:(gg + 1) * 128] for gg in range(16)]

    def cswap(a, b):
        return jnp.maximum(a, b), jnp.minimum(a, b)

    def sort8(v):
        net = [(0, 1), (2, 3), (4, 5), (6, 7),
               (0, 2), (1, 3), (4, 6), (5, 7),
               (1, 2), (5, 6), (0, 4), (3, 7),
               (1, 5), (2, 6),
               (1, 4), (3, 6),
               (2, 4), (3, 5),
               (3, 4)]
        v = list(v)
        for a, b in net:
            v[a], v[b] = cswap(v[a], v[b])
        return v

    lo = sort8(sl[0:8])
    hi = sort8(sl[8:16])
    top = [jnp.maximum(lo[i], hi[7 - i]) for i in range(8)]
    cand = jnp.concatenate(top, axis=1)

    m = None
    for t in range(_K):
        m = jnp.max(cand, axis=1, keepdims=True)
        if t < _K - 1:
            cand = jnp.where(cand == m, neg_inf, cand)
    mask8 = key0 >= m

    vs = [jnp.max(jnp.where(mask8, pt[o:o + 1, :], neg_inf), axis=1, keepdims=True)
          for o in range(_OUT_F)]
    v3 = jnp.concatenate(vs, axis=1)

    cr = _bdot(xr, wd) + b3
    bias_tile = jnp.concatenate([bias_ref[...]] * (_ROWS // _DEGREE), axis=0)
    o = v3 + cr + bias_tile
    out_ref[0] = _leaky(o)


@jax.jit
def kernel(tree0, tree1, tree2, tree3, tree4, tree5,
           W_root0, W_root1, W_root2, W_root3, W_root4, W_root5,
           W_branch, W_loop1, W_loop2, bias,
           conv1_w, conv1_b, conv2_w, conv2_b):
    full = lambda s: pl.BlockSpec(s, lambda n: (0,) * len(s))
    x = pl.pallas_call(
        _stage1_body,
        grid=(_NODE,),
        in_specs=[
            full(tree0.shape), full(tree1.shape), full(tree2.shape),
            full(tree3.shape), full(tree4.shape), full(tree5.shape),
            full(W_root0.shape), full(W_root1.shape), full(W_root2.shape),
            full(W_root3.shape), full(W_root4.shape), full(W_root5.shape),
            pl.BlockSpec((1, _IN_F, _DEGREE * _IN_F), lambda n: (n, 0, 0)),
            full(W_loop1.shape), full(W_loop2.shape),
        ],
        out_specs=pl.BlockSpec((_BATCH, _DEGREE, _OUT_F), lambda n: (0, n, 0)),
        out_shape=jax.ShapeDtypeStruct((_BATCH, _N, _OUT_F), jnp.float32),
        compiler_params=pltpu.CompilerParams(
            dimension_semantics=("arbitrary",),
        ),
    )(tree0, tree1, tree2, tree3, tree4, tree5,
      W_root0, W_root1, W_root2, W_root3, W_root4, W_root5,
      W_branch, W_loop1, W_loop2)

    full2 = lambda s: pl.BlockSpec(s, lambda b, r: (0,) * len(s))
    out = pl.pallas_call(
        _stage2_body,
        grid=(_BATCH, _N // _ROWS),
        in_specs=[
            pl.BlockSpec((1, _N, _OUT_F), lambda b, r: (b, 0, 0)),
            full2(conv1_w.shape),
            full2((1, 64)),
            full2(conv2_w.shape),
            full2((1, 3)),
            full2((_DEGREE, _OUT_F)),
        ],
        out_specs=pl.BlockSpec((1, _ROWS, _OUT_F), lambda b, r: (b, r, 0)),
        out_shape=jax.ShapeDtypeStruct((_BATCH, _N, _OUT_F), jnp.float32),
        scratch_shapes=[
            pltpu.VMEM((8, _N), jnp.float32),
        ],
        compiler_params=pltpu.CompilerParams(
            dimension_semantics=("arbitrary", "arbitrary"),
        ),
    )(x, conv1_w, conv1_b.reshape(1, 64), conv2_w, conv2_b.reshape(1, 3),
      bias.reshape(_DEGREE, _OUT_F))
    return out


# speedup vs baseline: 8.9626x; 1.0257x over previous
import jax
import jax.numpy as jnp
from jax.experimental import pallas as pl
from jax.experimental.pallas import tpu as pltpu

_BATCH = 16
_NODE = 32
_DEGREE = 64
_IN_F = 128
_OUT_F = 3
_N = _NODE * _DEGREE
_K = 8
_ROWS = 2048
_COUNTS = (1, 2, 4, 8, 16, 32)


def _leaky(x):
    return jnp.where(x >= 0, x, 0.2 * x)


def _bdot(a, b):
    return jnp.dot(a.astype(jnp.bfloat16), b.astype(jnp.bfloat16),
                   preferred_element_type=jnp.float32)


def _stage1_body(t0, t1, t2, t3, t4, t5,
                 wr0, wr1, wr2, wr3, wr4, wr5,
                 wb, wl1, wl2, x_out):
    n = pl.program_id(0)
    trees = (t0, t1, t2, t3, t4, t5)
    wroots = (wr0, wr1, wr2, wr3, wr4, wr5)

    root_n = jnp.zeros((_BATCH, _OUT_F), dtype=jnp.float32)
    sel_row = None
    for i in range(6):
        cnt = _COUNTS[i]
        rep = _NODE // cnt
        idx = n // rep
        tv = trees[i][...]
        cids = jax.lax.broadcasted_iota(jnp.int32, tv.shape, 1)
        row = jnp.sum(jnp.where(cids == idx, tv, 0.0), axis=1)
        if i == 5:
            sel_row = row
        root_n = root_n + _bdot(row, wroots[i][...])

    bmat = _bdot(sel_row, wb[0])
    bmat = _leaky(bmat)
    bmat = bmat.reshape(_BATCH * _DEGREE, _IN_F)
    h = _bdot(bmat, wl1[...])
    q = _bdot(h, wl2[...])
    q = q.reshape(_BATCH, _DEGREE, _OUT_F)
    x_out[...] = q + root_n[:, None, :]


def _stage2_body(x_ref, c1w, c1b, c2w, c2b, bias_ref, out_ref, aux):
    r = pl.program_id(1)

    xb = x_ref[0]
    xr = x_ref[0, pl.ds(r * _ROWS, _ROWS), :]

    w12 = _bdot(c1w[...], c2w[...])
    w1 = w12[0:3, :]
    wd = w12[3:6, :] - w1
    b3 = _bdot(c1b[...], c2w[...]) + c2b[...]

    @pl.when(r == 0)
    def _hoist():
        aux[0:3, :] = jax.lax.dot_general(w1, xb, (((0,), (1,)), ((), ())),
                                          preferred_element_type=jnp.float32,
                                          precision=jax.lax.Precision.HIGHEST)
        aux[3:4, :] = jax.lax.dot_general(jnp.ones((1, 3), jnp.float32), xb * xb,
                                          (((1,), (1,)), ((), ())),
                                          preferred_element_type=jnp.float32,
                                          precision=jax.lax.Precision.HIGHEST)

    pt = aux[0:3, :]
    sq_row = aux[3:4, :]

    g = jax.lax.dot_general(xr.astype(jnp.bfloat16), xb.astype(jnp.bfloat16),
                            (((1,), (1,)), ((), ())),
                            preferred_element_type=jnp.float32)
    inner = -2.0 * g
    sq_col = jnp.sum(xr * xr, axis=1, keepdims=True)
    key0 = (-sq_row) - inner - sq_col

    neg_inf = jnp.float32(-jnp.inf)

    sl = [key0[:, gg * 128:(gg + 1) * 128] for gg in range(16)]

    def cswap(a, b):
        return jnp.maximum(a, b), jnp.minimum(a, b)

    def sort8(v):
        net = [(0, 1), (2, 3), (4, 5), (6, 7),
               (0, 2), (1, 3), (4, 6), (5, 7),
               (1, 2), (5, 6), (0, 4), (3, 7),
               (1, 5), (2, 6),
               (1, 4), (3, 6),
               (2, 4), (3, 5),
               (3, 4)]
        v = list(v)
        for a, b in net:
            v[a], v[b] = cswap(v[a], v[b])
        return v

    lo = sort8(sl[0:8])
    hi = sort8(sl[8:16])
    top = [jnp.maximum(lo[i], hi[7 - i]) for i in range(8)]
    cand = jnp.concatenate(top, axis=1)

    m = None
    for t in range(_K):
        m = jnp.max(cand, axis=1, keepdims=True)
        if t < _K - 1:
            cand = jnp.where(cand == m, neg_inf, cand)
    mask8 = key0 >= m

    vs = [jnp.max(jnp.where(mask8, pt[o:o + 1, :], neg_inf), axis=1, keepdims=True)
          for o in range(_OUT_F)]
    v3 = jnp.concatenate(vs, axis=1)

    cr = _bdot(xr, wd) + b3
    bias_tile = jnp.concatenate([bias_ref[...]] * (_ROWS // _DEGREE), axis=0)
    o = v3 + cr + bias_tile
    out_ref[0] = _leaky(o)


@jax.jit
def kernel(tree0, tree1, tree2, tree3, tree4, tree5,
           W_root0, W_root1, W_root2, W_root3, W_root4, W_root5,
           W_branch, W_loop1, W_loop2, bias,
           conv1_w, conv1_b, conv2_w, conv2_b):
    full = lambda s: pl.BlockSpec(s, lambda n: (0,) * len(s))
    x = pl.pallas_call(
        _stage1_body,
        grid=(_NODE,),
        in_specs=[
            full(tree0.shape), full(tree1.shape), full(tree2.shape),
            full(tree3.shape), full(tree4.shape), full(tree5.shape),
            full(W_root0.shape), full(W_root1.shape), full(W_root2.shape),
            full(W_root3.shape), full(W_root4.shape), full(W_root5.shape),
            pl.BlockSpec((1, _IN_F, _DEGREE * _IN_F), lambda n: (n, 0, 0)),
            full(W_loop1.shape), full(W_loop2.shape),
        ],
        out_specs=pl.BlockSpec((_BATCH, _DEGREE, _OUT_F), lambda n: (0, n, 0)),
        out_shape=jax.ShapeDtypeStruct((_BATCH, _N, _OUT_F), jnp.float32),
        compiler_params=pltpu.CompilerParams(
            dimension_semantics=("arbitrary",),
        ),
    )(tree0, tree1, tree2, tree3, tree4, tree5,
      W_root0, W_root1, W_root2, W_root3, W_root4, W_root5,
      W_branch, W_loop1, W_loop2)

    full2 = lambda s: pl.BlockSpec(s, lambda b, r: (0,) * len(s))
    out = pl.pallas_call(
        _stage2_body,
        grid=(_BATCH, _N // _ROWS),
        in_specs=[
            pl.BlockSpec((1, _N, _OUT_F), lambda b, r: (b, 0, 0)),
            full2(conv1_w.shape),
            full2((1, 64)),
            full2(conv2_w.shape),
            full2((1, 3)),
            full2((_DEGREE, _OUT_F)),
        ],
        out_specs=pl.BlockSpec((1, _ROWS, _OUT_F), lambda b, r: (b, r, 0)),
        out_shape=jax.ShapeDtypeStruct((_BATCH, _N, _OUT_F), jnp.float32),
        scratch_shapes=[
            pltpu.VMEM((8, _N), jnp.float32),
        ],
        compiler_params=pltpu.CompilerParams(
            dimension_semantics=("arbitrary", "arbitrary"),
        ),
    )(x, conv1_w, conv1_b.reshape(1, 64), conv2_w, conv2_b.reshape(1, 3),
      bias.reshape(_DEGREE, _OUT_F))
    return out
